# baseline (speedup 1.0000x reference)
import functools

import jax
import jax.numpy as jnp
from jax import lax
from jax.experimental import pallas as pl
from jax.experimental.pallas import tpu as pltpu
from jax.experimental.pallas import tpu_sc as plsc

D_MODEL = 512
FONT_X = 128
FONT_Y = 128
VOCAB = 512
BATCH = 32
SEQ = 2048

NUM_CORES = 2
NUM_SUBCORES = 16
NUM_WORKERS = NUM_CORES * NUM_SUBCORES
LANES = 16

S_OWN = SEQ // NUM_WORKERS
ROWS = 32
NGBUF = 2
NWBUF = 4
STEPS = BATCH * S_OWN // ROWS


def _fuse_body(tok_ref, cxm_ref, cym_ref, cx1_ref, cy1_ref, out_ref):
    out_ref[0:FONT_X, :] = tok_ref[0:FONT_X, :] + cxm_ref[:, :] + cy1_ref[:, :]
    out_ref[FONT_X:FONT_X + FONT_Y, :] = (
        tok_ref[FONT_X:FONT_X + FONT_Y, :] + cx1_ref[:, :] + cym_ref[:, :])
    out_ref[FONT_X + FONT_Y:VOCAB, :] = (
        tok_ref[FONT_X + FONT_Y:VOCAB, :] + cx1_ref[:, :] + cy1_ref[:, :])


def _build_fused(token_table, coord_x_table, coord_y_table):
    cxm = coord_x_table[1:FONT_X + 1]
    cym = coord_y_table[1:FONT_Y + 1]
    cx1 = coord_x_table[1:2]
    cy1 = coord_y_table[1:2]
    return pl.pallas_call(
        _fuse_body,
        out_shape=jax.ShapeDtypeStruct((VOCAB, D_MODEL), jnp.float32),
    )(token_table, cxm, cym, cx1, cy1)


def _pack_table(fused):
    fb = fused.astype(jnp.bfloat16)
    fb = fb.reshape(VOCAB, D_MODEL // 32, 2, 16).transpose(0, 1, 3, 2)
    return lax.bitcast_convert_type(
        fb.reshape(VOCAB, D_MODEL // 2, 2), jnp.int32)


def _lookup_body(tok_hbm, fused_hbm, pos_hbm, out_hbm,
                 idx_all, pos_v, gbufs, wbufs, gsems, wsems, isem):
    wid = lax.axis_index("s") * NUM_CORES + lax.axis_index("c")
    s0 = wid * S_OWN
    pltpu.sync_copy(pos_hbm.at[pl.ds(s0, S_OWN)], pos_v)
    icps = [pltpu.async_copy(tok_hbm.at[pl.ds(b * SEQ + s0, S_OWN)],
                             idx_all.at[b], isem)
            for b in range(BATCH)]
    for c in icps:
        c.wait()

    def gather_start(t, p):
        idx_ref = idx_all.at[t // 2, pl.ds((t % 2) * ROWS, ROWS)]
        pltpu.async_copy(fused_hbm.at[idx_ref], gbufs[p], gsems[p])

    def wait_gather(p):
        pltpu.make_async_copy(
            fused_hbm.at[idx_all.at[0, pl.ds(0, ROWS)]],
            gbufs[p], gsems[p]).wait()

    def write_start(t, p):
        off = (t // 2) * SEQ + s0 + (t % 2) * ROWS
        pltpu.async_copy(wbufs[p], out_hbm.at[pl.ds(off, ROWS)], wsems[p])

    def wait_write(p):
        pltpu.make_async_copy(
            wbufs[p], out_hbm.at[pl.ds(0, ROWS)], wsems[p]).wait()

    gather_start(0, 0)

    def outer_body(g, carry):
        for ph in range(NWBUF):
            t = g * NWBUF + ph
            pg = ph % NGBUF
            if ph < NWBUF - 1:
                gather_start(t + 1, (pg + 1) % NGBUF)
            else:
                @pl.when(g < STEPS // NWBUF - 1)
                def _():
                    gather_start(t + 1, (pg + 1) % NGBUF)
            @pl.when(g >= 1)
            def _():
                wait_write(ph)
            wait_gather(pg)
            half = ph % 2

            def row_body(j, c2, _pg=pg, _ph=ph, _half=half):
                for k in range(D_MODEL // 32):
                    x = gbufs[_pg][j, pl.ds(k * LANES, LANES)]
                    e = plsc.bitcast(lax.shift_left(x, 16), jnp.float32)
                    o = plsc.bitcast(lax.bitwise_and(x, -65536), jnp.float32)
                    lo = pl.ds(k * 32, LANES)
                    hi = pl.ds(k * 32 + LANES, LANES)
                    prow = _half * ROWS + j
                    wbufs[_ph][j, lo] = e + pos_v[prow, lo]
                    wbufs[_ph][j, hi] = o + pos_v[prow, hi]
                return c2

            lax.fori_loop(0, ROWS, row_body, 0)
            write_start(t, ph)
        return carry

    lax.fori_loop(0, STEPS // NWBUF, outer_body, 0)
    for p in range(NWBUF):
        wait_write(p)


_lookup = functools.partial(
    pl.kernel,
    out_type=jax.ShapeDtypeStruct((BATCH * SEQ, D_MODEL), jnp.float32),
    mesh=plsc.VectorSubcoreMesh(
        core_axis_name="c", subcore_axis_name="s",
        num_cores=NUM_CORES, num_subcores=NUM_SUBCORES),
    compiler_params=pltpu.CompilerParams(needs_layout_passes=False),
    scratch_types=[
        pltpu.VMEM((BATCH, S_OWN), jnp.int32),
        pltpu.VMEM((S_OWN, D_MODEL), jnp.float32),
        [pltpu.VMEM((ROWS, D_MODEL // 2), jnp.int32) for _ in range(NGBUF)],
        [pltpu.VMEM((ROWS, D_MODEL), jnp.float32) for _ in range(NWBUF)],
        [pltpu.SemaphoreType.DMA for _ in range(NGBUF)],
        [pltpu.SemaphoreType.DMA for _ in range(NWBUF)],
        pltpu.SemaphoreType.DMA,
    ],
)(_lookup_body)


def kernel(font_tokens, token_table, coord_x_table, coord_y_table, pos_table):
    fused = _build_fused(token_table, coord_x_table, coord_y_table)
    packed = _pack_table(fused)
    tokens = font_tokens.astype(jnp.int32).reshape(BATCH * SEQ)
    out = _lookup(tokens, packed, pos_table)
    return out.reshape(BATCH, SEQ, D_MODEL)


# speedup vs baseline: 1.9299x; 1.9299x over previous
import functools

import jax
import jax.numpy as jnp
from jax import lax
from jax.experimental import pallas as pl
from jax.experimental.pallas import tpu as pltpu
from jax.experimental.pallas import tpu_sc as plsc

D_MODEL = 512
FONT_X = 128
FONT_Y = 128
VOCAB = 512
BATCH = 32
SEQ = 2048

NUM_CORES = 2
NUM_SUBCORES = 16
NUM_WORKERS = NUM_CORES * NUM_SUBCORES
LANES = 16

S_OWN = SEQ // NUM_WORKERS
ROWS = 32
NGBUF = 2
NWBUF = 4
STEPS = BATCH * S_OWN // ROWS


def _fuse_body(tok_ref, cxm_ref, cym_ref, cx1_ref, cy1_ref, out_ref):
    out_ref[0:FONT_X, :] = tok_ref[0:FONT_X, :] + cxm_ref[:, :] + cy1_ref[:, :]
    out_ref[FONT_X:FONT_X + FONT_Y, :] = (
        tok_ref[FONT_X:FONT_X + FONT_Y, :] + cx1_ref[:, :] + cym_ref[:, :])
    out_ref[FONT_X + FONT_Y:VOCAB, :] = (
        tok_ref[FONT_X + FONT_Y:VOCAB, :] + cx1_ref[:, :] + cy1_ref[:, :])


def _build_fused(token_table, coord_x_table, coord_y_table):
    cxm = coord_x_table[1:FONT_X + 1]
    cym = coord_y_table[1:FONT_Y + 1]
    cx1 = coord_x_table[1:2]
    cy1 = coord_y_table[1:2]
    return pl.pallas_call(
        _fuse_body,
        out_shape=jax.ShapeDtypeStruct((VOCAB, D_MODEL), jnp.float32),
    )(token_table, cxm, cym, cx1, cy1)


def _pack_table(fused):
    fb = fused.astype(jnp.bfloat16)
    fb = fb.reshape(VOCAB, D_MODEL // 32, 2, 16).transpose(0, 1, 3, 2)
    return lax.bitcast_convert_type(
        fb.reshape(VOCAB, D_MODEL // 2, 2), jnp.int32)


def _lookup_body(tok_hbm, fused_hbm, pos_hbm, out_hbm,
                 idx_all, pos_v, gbufs, wbufs, gsems, wsems, isem):
    wid = lax.axis_index("s") * NUM_CORES + lax.axis_index("c")
    s0 = wid * S_OWN
    pltpu.sync_copy(pos_hbm.at[pl.ds(s0, S_OWN)], pos_v)
    icps = [pltpu.async_copy(tok_hbm.at[pl.ds(b * SEQ + s0, S_OWN)],
                             idx_all.at[b], isem)
            for b in range(BATCH)]
    for c in icps:
        c.wait()

    def gather_start(t, p):
        idx_ref = idx_all.at[t // 2, pl.ds((t % 2) * ROWS, ROWS)]
        pltpu.async_copy(fused_hbm.at[idx_ref], gbufs[p], gsems[p])

    def wait_gather(p):
        pltpu.make_async_copy(
            fused_hbm.at[idx_all.at[0, pl.ds(0, ROWS)]],
            gbufs[p], gsems[p]).wait()

    def write_start(t, p):
        off = (t // 2) * SEQ + s0 + (t % 2) * ROWS
        pltpu.async_copy(wbufs[p], out_hbm.at[pl.ds(off, ROWS)], wsems[p])

    def wait_write(p):
        pltpu.make_async_copy(
            wbufs[p], out_hbm.at[pl.ds(0, ROWS)], wsems[p]).wait()

    gather_start(0, 0)

    def outer_body(g, carry):
        for ph in range(NWBUF):
            t = g * NWBUF + ph
            pg = ph % NGBUF
            if ph < NWBUF - 1:
                gather_start(t + 1, (pg + 1) % NGBUF)
            else:
                @pl.when(g < STEPS // NWBUF - 1)
                def _():
                    gather_start(t + 1, (pg + 1) % NGBUF)
            @pl.when(g >= 1)
            def _():
                wait_write(ph)
            wait_gather(pg)
            half = ph % 2

            @plsc.parallel_loop(0, ROWS, 1, unroll=2)
            def row_body(j, _pg=pg, _ph=ph, _half=half):
                for k in range(D_MODEL // 32):
                    x = gbufs[_pg][j, pl.ds(k * LANES, LANES)]
                    e = plsc.bitcast(lax.shift_left(x, 16), jnp.float32)
                    o = plsc.bitcast(lax.bitwise_and(x, -65536), jnp.float32)
                    lo = pl.ds(k * 32, LANES)
                    hi = pl.ds(k * 32 + LANES, LANES)
                    prow = _half * ROWS + j
                    wbufs[_ph][j, lo] = e + pos_v[prow, lo]
                    wbufs[_ph][j, hi] = o + pos_v[prow, hi]
            write_start(t, ph)
        return carry

    lax.fori_loop(0, STEPS // NWBUF, outer_body, 0)
    for p in range(NWBUF):
        wait_write(p)


_lookup = functools.partial(
    pl.kernel,
    out_type=jax.ShapeDtypeStruct((BATCH * SEQ, D_MODEL), jnp.float32),
    mesh=plsc.VectorSubcoreMesh(
        core_axis_name="c", subcore_axis_name="s",
        num_cores=NUM_CORES, num_subcores=NUM_SUBCORES),
    compiler_params=pltpu.CompilerParams(needs_layout_passes=False),
    scratch_types=[
        pltpu.VMEM((BATCH, S_OWN), jnp.int32),
        pltpu.VMEM((S_OWN, D_MODEL), jnp.float32),
        [pltpu.VMEM((ROWS, D_MODEL // 2), jnp.int32) for _ in range(NGBUF)],
        [pltpu.VMEM((ROWS, D_MODEL), jnp.float32) for _ in range(NWBUF)],
        [pltpu.SemaphoreType.DMA for _ in range(NGBUF)],
        [pltpu.SemaphoreType.DMA for _ in range(NWBUF)],
        pltpu.SemaphoreType.DMA,
    ],
)(_lookup_body)


def kernel(font_tokens, token_table, coord_x_table, coord_y_table, pos_table):
    fused = _build_fused(token_table, coord_x_table, coord_y_table)
    packed = _pack_table(fused)
    tokens = font_tokens.astype(jnp.int32).reshape(BATCH * SEQ)
    out = _lookup(tokens, packed, pos_table)
    return out.reshape(BATCH, SEQ, D_MODEL)
